# Initial kernel scaffold; baseline (speedup 1.0000x reference)
import jax
import jax.numpy as jnp
from jax.experimental import pallas as pl


def kernel(batch, X, H, E_idx, E, Z, params):
    raise NotImplementedError("write your pallas kernel here")


# speedup vs baseline: 11.1720x; 11.1720x over previous
import functools

import jax
import jax.numpy as jnp
import numpy as np
from jax import lax
from jax.experimental import pallas as pl
from jax.experimental.pallas import tpu as pltpu
from jax.experimental.pallas import tpu_sc as plsc

N_NODES = 50000
N_EDGES = 800000
D_H = 64
D_E = 32
HEADS = 4
D_HEAD = D_H // HEADS
N_GRAPHS = 64
XW = 16
NC, NS = 2, 16
NW = NC * NS


def _silu(x):
    return x * jax.nn.sigmoid(x)


def _mesh():
    return plsc.VectorSubcoreMesh(
        core_axis_name="c", subcore_axis_name="s", num_cores=NC, num_subcores=NS
    )


def _gather_phase_a(H, Xp, src, dst):
    M = N_EDGES
    C = 200
    per_tile = M // NW
    iters = per_tile // C
    f32 = jnp.float32

    @functools.partial(
        pl.kernel,
        out_type=[
            jax.ShapeDtypeStruct((M, D_H), f32),
            jax.ShapeDtypeStruct((M, D_H), f32),
            jax.ShapeDtypeStruct((M, XW), f32),
            jax.ShapeDtypeStruct((M, XW), f32),
        ],
        mesh=_mesh(),
        compiler_params=pltpu.CompilerParams(use_tc_tiling_on_sc=False),
        scratch_types=[
            pltpu.VMEM((C,), jnp.int32),
            pltpu.VMEM((C,), jnp.int32),
            pltpu.VMEM((C, D_H), f32),
            pltpu.VMEM((C, D_H), f32),
            pltpu.VMEM((C, XW), f32),
            pltpu.VMEM((C, XW), f32),
            pltpu.SemaphoreType.DMA,
        ],
    )
    def k(h_hbm, x_hbm, src_hbm, dst_hbm, hd_o, hs_o, xs_o, xd_o,
          sbuf, dbuf, hdb, hsb, xsb, xdb, sem):
        c = lax.axis_index("c")
        s = lax.axis_index("s")
        wid = s * NC + c

        def body(kk, carry):
            base = wid * per_tile + kk * C
            pltpu.sync_copy(src_hbm.at[pl.ds(base, C)], sbuf)
            pltpu.sync_copy(dst_hbm.at[pl.ds(base, C)], dbuf)
            c1 = pltpu.async_copy(h_hbm.at[dbuf], hdb, sem)
            c2 = pltpu.async_copy(h_hbm.at[sbuf], hsb, sem)
            c3 = pltpu.async_copy(x_hbm.at[sbuf], xsb, sem)
            c4 = pltpu.async_copy(x_hbm.at[dbuf], xdb, sem)
            c1.wait()
            c2.wait()
            c3.wait()
            c4.wait()
            pltpu.sync_copy(hdb, hd_o.at[pl.ds(base, C)])
            pltpu.sync_copy(hsb, hs_o.at[pl.ds(base, C)])
            pltpu.sync_copy(xsb, xs_o.at[pl.ds(base, C)])
            pltpu.sync_copy(xdb, xd_o.at[pl.ds(base, C)])
            return carry

        lax.fori_loop(0, iters, body, 0)

    return k(H, Xp, src, dst)


def _gather_phase_b(q, KV, src, dst):
    M = N_EDGES
    C = 200
    per_tile = M // NW
    iters = per_tile // C
    f32 = jnp.float32

    @functools.partial(
        pl.kernel,
        out_type=[
            jax.ShapeDtypeStruct((M, D_H), f32),
            jax.ShapeDtypeStruct((M, 2 * D_H), f32),
        ],
        mesh=_mesh(),
        compiler_params=pltpu.CompilerParams(use_tc_tiling_on_sc=False),
        scratch_types=[
            pltpu.VMEM((C,), jnp.int32),
            pltpu.VMEM((C,), jnp.int32),
            pltpu.VMEM((C, D_H), f32),
            pltpu.VMEM((C, 2 * D_H), f32),
            pltpu.SemaphoreType.DMA,
        ],
    )
    def k(q_hbm, kv_hbm, src_hbm, dst_hbm, qd_o, kvs_o, sbuf, dbuf, qdb, kvb, sem):
        c = lax.axis_index("c")
        s = lax.axis_index("s")
        wid = s * NC + c

        def body(kk, carry):
            base = wid * per_tile + kk * C
            pltpu.sync_copy(src_hbm.at[pl.ds(base, C)], sbuf)
            pltpu.sync_copy(dst_hbm.at[pl.ds(base, C)], dbuf)
            c1 = pltpu.async_copy(q_hbm.at[dbuf], qdb, sem)
            c2 = pltpu.async_copy(kv_hbm.at[sbuf], kvb, sem)
            c1.wait()
            c2.wait()
            pltpu.sync_copy(qdb, qd_o.at[pl.ds(base, C)])
            pltpu.sync_copy(kvb, kvs_o.at[pl.ds(base, C)])
            return carry

        lax.fori_loop(0, iters, body, 0)

    return k(q, KV, src, dst)


def _gather_rows16(tbl, idx):
    M = N_EDGES
    C = 1000
    per_tile = M // NW
    iters = per_tile // C
    f32 = jnp.float32

    @functools.partial(
        pl.kernel,
        out_type=jax.ShapeDtypeStruct((M, XW), f32),
        mesh=_mesh(),
        compiler_params=pltpu.CompilerParams(use_tc_tiling_on_sc=False),
        scratch_types=[
            pltpu.VMEM((C,), jnp.int32),
            pltpu.VMEM((C, XW), f32),
            pltpu.SemaphoreType.DMA,
        ],
    )
    def k(t_hbm, idx_hbm, o_hbm, ibuf, rbuf, sem):
        c = lax.axis_index("c")
        s = lax.axis_index("s")
        wid = s * NC + c

        def body(kk, carry):
            base = wid * per_tile + kk * C
            pltpu.sync_copy(idx_hbm.at[pl.ds(base, C)], ibuf)
            pltpu.async_copy(t_hbm.at[ibuf], rbuf, sem).wait()
            pltpu.sync_copy(rbuf, o_hbm.at[pl.ds(base, C)])
            return carry

        lax.fori_loop(0, iters, body, 0)

    return k(tbl, idx)


def _scatter_add3(PS, dst, W, zblk):
    M = N_EDGES
    N = N_NODES
    C = 1000
    per_core = M // NC
    per_tile = per_core // NS
    iters = per_tile // C
    ZCH = 1000
    nz = N // ZCH
    zrounds = (nz + NS - 1) // NS
    f32 = jnp.float32
    NP = len(PS)

    @functools.partial(
        pl.kernel,
        out_type=[jax.ShapeDtypeStruct((NC * N, W), f32) for _ in range(NP)],
        mesh=_mesh(),
        compiler_params=pltpu.CompilerParams(use_tc_tiling_on_sc=False),
        scratch_types=[
            pltpu.VMEM((C, W), f32),
            pltpu.VMEM((C,), jnp.int32),
            pltpu.VMEM_SHARED((N, W), f32),
        ],
    )
    def k(*refs):
        p_hbms = refs[:NP]
        dst_hbm = refs[NP]
        z_hbm = refs[NP + 1]
        o_hbms = refs[NP + 2:NP + 2 + NP]
        pbuf, ibuf, acc = refs[NP + 2 + NP:]
        c = lax.axis_index("c")
        s = lax.axis_index("s")
        for p_hbm, o_hbm in zip(p_hbms, o_hbms):
            for g in range(zrounds):
                cid = g * NS + s

                @pl.when(cid < nz)
                def _():
                    pltpu.sync_copy(z_hbm, acc.at[pl.ds(cid * ZCH, ZCH)])

            plsc.subcore_barrier()

            def body(kk, carry):
                base = c * per_core + s * per_tile + kk * C
                pltpu.sync_copy(dst_hbm.at[pl.ds(base, C)], ibuf)
                pltpu.sync_copy(p_hbm.at[pl.ds(base, C)], pbuf)
                pltpu.sync_copy(pbuf, acc.at[ibuf], add=True)
                return carry

            lax.fori_loop(0, iters, body, 0)
            plsc.subcore_barrier()
            for g in range(zrounds):
                cid = g * NS + s

                @pl.when(cid < nz)
                def _():
                    pltpu.sync_copy(
                        acc.at[pl.ds(cid * ZCH, ZCH)],
                        o_hbm.at[pl.ds(c * N + cid * ZCH, ZCH)],
                    )

            plsc.subcore_barrier()

    out = k(*PS, dst, zblk)
    return list(out) if isinstance(out, (list, tuple)) else [out]


_REDGE = 2000
_RNODE = 2000


def _edge_mlp(Hd, Hs, Xs, Xd, E, Wm1, bm1, Wm2, bm2, Wx8, bx8, We, be):
    R = _REDGE
    G = N_EDGES // R
    f32 = jnp.float32

    def body(hd, hs, xs, xd, e, wm1, bm1r, wm2, bm2r, wx, bxr, we, ber,
             p0_o, p1_o, p2_o, e1_o):
        rel = xs[...] - xd[...]
        d2 = jnp.sum(rel * rel, axis=1, keepdims=True)
        feat = jnp.concatenate([hd[...], hs[...], d2, e[...]], axis=1)
        m = _silu(jnp.dot(feat, wm1[...], preferred_element_type=f32) + bm1r[...])
        m_ij = _silu(jnp.dot(m, wm2[...], preferred_element_type=f32) + bm2r[...])
        coef = jnp.tanh(jnp.dot(m_ij, wx[...], preferred_element_type=f32) + bxr[...])[:, 0:1]
        rc = jnp.concatenate(
            [rel[:, 0:3] * coef, jnp.ones((R, 1), f32), jnp.zeros((R, 4), f32)],
            axis=1,
        )
        p0_o[...] = m_ij[:, :24]
        p1_o[...] = m_ij[:, 24:48]
        p2_o[...] = jnp.concatenate([m_ij[:, 48:], rc], axis=1)
        e1_o[...] = e[...] + jnp.dot(
            jnp.concatenate([e[...], m_ij], axis=1), we[...],
            preferred_element_type=f32) + ber[...]

    row = lambda i: (i, 0)
    fix = lambda i: (0, 0)
    return pl.pallas_call(
        body,
        grid=(G,),
        in_specs=[
            pl.BlockSpec((R, D_H), row),
            pl.BlockSpec((R, D_H), row),
            pl.BlockSpec((R, XW), row),
            pl.BlockSpec((R, XW), row),
            pl.BlockSpec((R, D_E), row),
            pl.BlockSpec((2 * D_H + 1 + D_E, D_H), fix),
            pl.BlockSpec((1, D_H), fix),
            pl.BlockSpec((D_H, D_H), fix),
            pl.BlockSpec((1, D_H), fix),
            pl.BlockSpec((D_H, 8), fix),
            pl.BlockSpec((1, 8), fix),
            pl.BlockSpec((D_E + D_H, D_E), fix),
            pl.BlockSpec((1, D_E), fix),
        ],
        out_specs=[
            pl.BlockSpec((R, 24), row),
            pl.BlockSpec((R, 24), row),
            pl.BlockSpec((R, 24), row),
            pl.BlockSpec((R, D_E), row),
        ],
        out_shape=[
            jax.ShapeDtypeStruct((N_EDGES, 24), f32),
            jax.ShapeDtypeStruct((N_EDGES, 24), f32),
            jax.ShapeDtypeStruct((N_EDGES, 24), f32),
            jax.ShapeDtypeStruct((N_EDGES, D_E), f32),
        ],
    )(Hd, Hs, Xs, Xd, E, Wm1, bm1, Wm2, bm2, Wx8, bx8, We, be)


def _node_update(H, Xp, S0, S1, S2, Wh1, bh1, Wh2, bh2, Wq, Wk, Wv):
    R = _RNODE
    G = N_NODES // R
    f32 = jnp.float32

    def body(h, xp, s0a, s0b, s1a, s1b, s2a, s2b, wh1, bh1r, wh2, bh2r, wq, wk, wv,
             h1_o, x1_o, q_o, kv_o, inv_o):
        s0 = s0a[...] + s0b[...]
        s1 = s1a[...] + s1b[...]
        s2 = s2a[...] + s2b[...]
        agg = jnp.concatenate([s0, s1, s2[:, :16]], axis=1)
        h1 = h[...] + jnp.dot(
            _silu(jnp.dot(jnp.concatenate([h[...], agg], axis=1), wh1[...],
                          preferred_element_type=f32) + bh1r[...]),
            wh2[...], preferred_element_type=f32) + bh2r[...]
        deg = s2[:, 19:20]
        inv = 1.0 / (deg + 1.0)
        upd = jnp.concatenate([s2[:, 16:19] * inv, jnp.zeros((R, XW - 3), f32)], axis=1)
        h1_o[...] = h1
        x1_o[...] = xp[...] + upd
        q_o[...] = jnp.dot(h1, wq[...], preferred_element_type=f32)
        kv_o[...] = jnp.concatenate(
            [jnp.dot(h1, wk[...], preferred_element_type=f32),
             jnp.dot(h1, wv[...], preferred_element_type=f32)], axis=1)
        inv_o[...] = jnp.concatenate([inv, jnp.zeros((R, 7), f32)], axis=1)

    row = lambda i: (i, 0)
    fix = lambda i: (0, 0)
    s40row = lambda i: (i, 0)
    s40row_b = lambda i: (i, 0)
    return pl.pallas_call(
        body,
        grid=(G,),
        in_specs=[
            pl.BlockSpec((R, D_H), row),
            pl.BlockSpec((R, XW), row),
            pl.BlockSpec((R, 24), s40row),
            pl.BlockSpec((R, 24), s40row_b),
            pl.BlockSpec((R, 24), s40row),
            pl.BlockSpec((R, 24), s40row_b),
            pl.BlockSpec((R, 24), s40row),
            pl.BlockSpec((R, 24), s40row_b),
            pl.BlockSpec((2 * D_H, D_H), fix),
            pl.BlockSpec((1, D_H), fix),
            pl.BlockSpec((D_H, D_H), fix),
            pl.BlockSpec((1, D_H), fix),
            pl.BlockSpec((D_H, D_H), fix),
            pl.BlockSpec((D_H, D_H), fix),
            pl.BlockSpec((D_H, D_H), fix),
        ],
        out_specs=[
            pl.BlockSpec((R, D_H), row),
            pl.BlockSpec((R, XW), row),
            pl.BlockSpec((R, D_H), row),
            pl.BlockSpec((R, 2 * D_H), row),
            pl.BlockSpec((R, 8), row),
        ],
        out_shape=[
            jax.ShapeDtypeStruct((N_NODES, D_H), f32),
            jax.ShapeDtypeStruct((N_NODES, XW), f32),
            jax.ShapeDtypeStruct((N_NODES, D_H), f32),
            jax.ShapeDtypeStruct((N_NODES, 2 * D_H), f32),
            jax.ShapeDtypeStruct((N_NODES, 8), f32),
        ],
    )(H, Xp, S0[:N_NODES], S0[N_NODES:], S1[:N_NODES], S1[N_NODES:],
      S2[:N_NODES], S2[N_NODES:], Wh1, bh1, Wh2, bh2, Wq, Wk, Wv)


def _logits(qd, KVs, E1, Shead, Web8):
    R = _REDGE
    G = N_EDGES // R
    f32 = jnp.float32

    def body(q, kv, e1, sh, web, l_o, g_o):
        i = pl.program_id(0)
        prod = q[...] * kv[:, :D_H]
        l = (jnp.dot(prod, sh[...], preferred_element_type=f32) * (1.0 / np.sqrt(D_HEAD))
             + jnp.dot(e1[...], web[...], preferred_element_type=f32))
        l_o[...] = l
        bm = jnp.max(l, axis=0)

        @pl.when(i == 0)
        def _():
            g_o[...] = jnp.full((8, 128), -1e30, f32)

        g_o[...] = jnp.maximum(g_o[...], jnp.broadcast_to(bm[:, None], (8, 128)))

    row = lambda i: (i, 0)
    fix = lambda i: (0, 0)
    return pl.pallas_call(
        body,
        grid=(G,),
        in_specs=[
            pl.BlockSpec((R, D_H), row),
            pl.BlockSpec((R, 2 * D_H), row),
            pl.BlockSpec((R, D_E), row),
            pl.BlockSpec((D_H, 8), fix),
            pl.BlockSpec((D_E, 8), fix),
        ],
        out_specs=[pl.BlockSpec((R, 8), row), pl.BlockSpec((8, 128), fix)],
        out_shape=[
            jax.ShapeDtypeStruct((N_EDGES, 8), f32),
            jax.ShapeDtypeStruct((8, 128), f32),
        ],
    )(qd, KVs, E1, Shead, Web8)


def _alpha(logits, gmax):
    R = _REDGE
    G = N_EDGES // R
    f32 = jnp.float32

    def body(l, g, a_o):
        gv = g[:, 0:1]
        a = jnp.exp(l[...] - gv[:, 0][None, :])
        mask = lax.broadcasted_iota(jnp.int32, (R, XW), 1) < HEADS
        a_o[...] = jnp.where(mask, jnp.concatenate([a, jnp.zeros((R, 8), f32)], axis=1), 0.0)

    row = lambda i: (i, 0)
    fix = lambda i: (0, 0)
    return pl.pallas_call(
        body,
        grid=(G,),
        in_specs=[pl.BlockSpec((R, 8), row), pl.BlockSpec((8, 128), fix)],
        out_specs=pl.BlockSpec((R, XW), row),
        out_shape=jax.ShapeDtypeStruct((N_EDGES, XW), f32),
    )(logits, gmax)


def _denr(D16):
    R = _RNODE
    G = N_NODES // R
    f32 = jnp.float32

    def body(a, b, o):
        o[...] = 1.0 / (a[...] + b[...] + 1e-9)

    return pl.pallas_call(
        body,
        grid=(G,),
        in_specs=[
            pl.BlockSpec((R, XW), lambda i: (i, 0)),
            pl.BlockSpec((R, XW), lambda i: (i, 0)),
        ],
        out_specs=pl.BlockSpec((R, XW), lambda i: (i, 0)),
        out_shape=jax.ShapeDtypeStruct((N_NODES, XW), f32),
    )(D16[:N_NODES], D16[N_NODES:])


def _attn_msgs(alpha16, denrd, KVs, Xs, Xd, E1, Ehead, We2, be2):
    R = _REDGE
    G = N_EDGES // R
    f32 = jnp.float32

    def body(a, dr, kv, xs, xd, e1, eh, we2, be2r, p0_o, p1_o, p2_o, e2_o):
        attn = a[:, :HEADS] * dr[:, :HEADS]
        xw = jnp.mean(attn, axis=1, keepdims=True)
        rel = xs[...] - xd[...]
        relxw = jnp.concatenate([rel[:, 0:3] * xw, jnp.zeros((R, 5), f32)], axis=1)
        av = kv[:, D_H:] * jnp.dot(attn, eh[...], preferred_element_type=f32)
        p0_o[...] = av[:, :24]
        p1_o[...] = av[:, 24:48]
        p2_o[...] = jnp.concatenate([av[:, 48:], relxw], axis=1)
        e2_o[...] = e1[...] + jnp.dot(
            jnp.concatenate([e1[...], attn], axis=1), we2[...],
            preferred_element_type=f32) + be2r[...]

    row = lambda i: (i, 0)
    fix = lambda i: (0, 0)
    return pl.pallas_call(
        body,
        grid=(G,),
        in_specs=[
            pl.BlockSpec((R, XW), row),
            pl.BlockSpec((R, XW), row),
            pl.BlockSpec((R, 2 * D_H), row),
            pl.BlockSpec((R, XW), row),
            pl.BlockSpec((R, XW), row),
            pl.BlockSpec((R, D_E), row),
            pl.BlockSpec((HEADS, D_H), fix),
            pl.BlockSpec((D_E + HEADS, D_E), fix),
            pl.BlockSpec((1, D_E), fix),
        ],
        out_specs=[
            pl.BlockSpec((R, 24), row),
            pl.BlockSpec((R, 24), row),
            pl.BlockSpec((R, 24), row),
            pl.BlockSpec((R, D_E), row),
        ],
        out_shape=[
            jax.ShapeDtypeStruct((N_EDGES, 24), f32),
            jax.ShapeDtypeStruct((N_EDGES, 24), f32),
            jax.ShapeDtypeStruct((N_EDGES, 24), f32),
            jax.ShapeDtypeStruct((N_EDGES, D_E), f32),
        ],
    )(alpha16, denrd, KVs, Xs, Xd, E1, Ehead, We2, be2)


def _final_nodes(H1, T0, T1, T2, inv8, X1, Z, batch3, Wo, Wz, bz):
    R = _RNODE
    G = N_NODES // R
    f32 = jnp.float32

    def body(h1, s0a, s0b, s1a, s1b, s2a, s2b, inv, x1, z, b3, wo, wz, bzr,
             h2_o, z1_o, x2_o, gs_o):
        i = pl.program_id(0)
        s0 = s0a[...] + s0b[...]
        s1 = s1a[...] + s1b[...]
        s2 = s2a[...] + s2b[...]
        agg2 = jnp.concatenate([s0, s1, s2[:, :16]], axis=1)
        h2 = h1[...] + jnp.dot(agg2, wo[...], preferred_element_type=f32)
        h2_o[...] = h2
        z1_o[...] = z[...] + _silu(jnp.dot(h2, wz[...], preferred_element_type=f32) + bzr[...])
        upd = jnp.concatenate(
            [s2[:, 16:19] * inv[:, 0:1], jnp.zeros((R, XW - 3), f32)], axis=1)
        x2 = x1[...] + upd
        x2_o[...] = x2
        lanes = lax.broadcasted_iota(jnp.int32, (R, XW), 1)
        x2c = jnp.where(lanes == 3, 1.0, x2)
        b = b3[0, 0, :]
        gids = lax.broadcasted_iota(jnp.int32, (R, N_GRAPHS), 1)
        onehot = (jnp.broadcast_to(b[:, None], (R, N_GRAPHS)) == gids).astype(f32)
        blk = lax.dot_general(onehot, x2c, (((0,), (0,)), ((), ())),
                              preferred_element_type=f32)

        @pl.when(i == 0)
        def _():
            gs_o[...] = jnp.zeros((N_GRAPHS, XW), f32)

        gs_o[...] = gs_o[...] + blk

    row = lambda i: (i, 0)
    fix = lambda i: (0, 0)
    return pl.pallas_call(
        body,
        grid=(G,),
        in_specs=[
            pl.BlockSpec((R, D_H), row),
            pl.BlockSpec((R, 24), row),
            pl.BlockSpec((R, 24), row),
            pl.BlockSpec((R, 24), row),
            pl.BlockSpec((R, 24), row),
            pl.BlockSpec((R, 24), row),
            pl.BlockSpec((R, 24), row),
            pl.BlockSpec((R, 8), row),
            pl.BlockSpec((R, XW), row),
            pl.BlockSpec((R, D_H), row),
            pl.BlockSpec((1, 1, R), lambda i: (i, 0, 0)),
            pl.BlockSpec((D_H, D_H), fix),
            pl.BlockSpec((D_H, D_H), fix),
            pl.BlockSpec((1, D_H), fix),
        ],
        out_specs=[
            pl.BlockSpec((R, D_H), row),
            pl.BlockSpec((R, D_H), row),
            pl.BlockSpec((R, XW), row),
            pl.BlockSpec((N_GRAPHS, XW), fix),
        ],
        out_shape=[
            jax.ShapeDtypeStruct((N_NODES, D_H), f32),
            jax.ShapeDtypeStruct((N_NODES, D_H), f32),
            jax.ShapeDtypeStruct((N_NODES, XW), f32),
            jax.ShapeDtypeStruct((N_GRAPHS, XW), f32),
        ],
    )(H1, T0[:N_NODES], T0[N_NODES:], T1[:N_NODES], T1[N_NODES:],
      T2[:N_NODES], T2[N_NODES:], inv8, X1, Z, batch3, Wo, Wz, bz)


def _center(X2, batch3, gsums):
    R = _RNODE
    G = N_NODES // R
    f32 = jnp.float32

    def body(x2, b3, gs, o):
        mean = gs[...] * (1.0 / (gs[:, 3:4] + 1e-9))
        b = b3[0, 0, :]
        gids = lax.broadcasted_iota(jnp.int32, (R, N_GRAPHS), 1)
        onehot = (jnp.broadcast_to(b[:, None], (R, N_GRAPHS)) == gids).astype(f32)
        o[...] = x2[...] - jnp.dot(onehot, mean, preferred_element_type=f32)

    return pl.pallas_call(
        body,
        grid=(G,),
        in_specs=[
            pl.BlockSpec((R, XW), lambda i: (i, 0)),
            pl.BlockSpec((1, 1, R), lambda i: (i, 0, 0)),
            pl.BlockSpec((N_GRAPHS, XW), lambda i: (0, 0)),
        ],
        out_specs=pl.BlockSpec((R, XW), lambda i: (i, 0)),
        out_shape=jax.ShapeDtypeStruct((N_NODES, XW), f32),
    )(X2, batch3, gsums)


def kernel(batch, X, H, E_idx, E, Z, params):
    p = params
    f32 = jnp.float32
    src = E_idx[0].astype(jnp.int32)
    dst = E_idx[1].astype(jnp.int32)
    Xp = jnp.pad(X.astype(f32), ((0, 0), (0, XW - 3)))
    batch3 = batch.astype(jnp.int32).reshape(N_NODES // _RNODE, 1, _RNODE)

    z24 = jnp.zeros((1000, 24), f32)
    z16 = jnp.zeros((1000, XW), f32)

    bm1 = p['b_m1'].reshape(1, D_H)
    bm2 = p['b_m2'].reshape(1, D_H)
    Wx8 = jnp.pad(p['W_x'], ((0, 0), (0, 7)))
    bx8 = jnp.pad(p['b_x'].reshape(1, 1), ((0, 0), (0, 7)))
    be = p['b_e'].reshape(1, D_E)
    bh1 = p['b_h1'].reshape(1, D_H)
    bh2 = p['b_h2'].reshape(1, D_H)
    be2 = p['b_e2'].reshape(1, D_E)
    bz = p['b_z'].reshape(1, D_H)
    Web8 = jnp.pad(p['W_eb'], ((0, 0), (0, 8 - HEADS)))
    shead = np.zeros((D_H, 8), np.float32)
    for h in range(HEADS):
        shead[h * D_HEAD:(h + 1) * D_HEAD, h] = 1.0
    Shead = jnp.asarray(shead)
    ehead = np.zeros((HEADS, D_H), np.float32)
    for h in range(HEADS):
        ehead[h, h * D_HEAD:(h + 1) * D_HEAD] = 1.0
    Ehead = jnp.asarray(ehead)

    Hd, Hs, Xs, Xd = _gather_phase_a(H, Xp, src, dst)
    P0, P1, P2, E1 = _edge_mlp(Hd, Hs, Xs, Xd, E, p['W_m1'], bm1, p['W_m2'], bm2,
                               Wx8, bx8, p['W_e'], be)
    S0, S1, S2 = _scatter_add3([P0, P1, P2], dst, 24, z24)
    H1, X1, q, KV, inv8 = _node_update(H, Xp, S0, S1, S2, p['W_h1'], bh1,
                                       p['W_h2'], bh2, p['W_q'], p['W_k'], p['W_v'])
    qd, KVs = _gather_phase_b(q, KV, src, dst)
    logits, gmax = _logits(qd, KVs, E1, Shead, Web8)
    alpha16 = _alpha(logits, gmax)
    D16, = _scatter_add3([alpha16], dst, XW, z16)
    denr = _denr(D16)
    denrd = _gather_rows16(denr, dst)
    Q0, Q1, Q2, E2 = _attn_msgs(alpha16, denrd, KVs, Xs, Xd, E1, Ehead,
                                p['W_e2'], be2)
    T0, T1, T2 = _scatter_add3([Q0, Q1, Q2], dst, 24, z24)
    H2, Z1, X2, gsums = _final_nodes(H1, T0, T1, T2, inv8, X1, Z, batch3,
                                     p['W_o'], p['W_z'], bz)
    Xout = _center(X2, batch3, gsums)
    return (Xout[:, :3], H2, E2, Z1)


# speedup vs baseline: 11.5751x; 1.0361x over previous
import functools

import jax
import jax.numpy as jnp
import numpy as np
from jax import lax
from jax.experimental import pallas as pl
from jax.experimental.pallas import tpu as pltpu
from jax.experimental.pallas import tpu_sc as plsc

N_NODES = 50000
N_EDGES = 800000
D_H = 64
D_E = 32
HEADS = 4
D_HEAD = D_H // HEADS
N_GRAPHS = 64
XW = 16
NC, NS = 2, 16
NW = NC * NS


def _silu(x):
    return x * jax.nn.sigmoid(x)


def _mesh():
    return plsc.VectorSubcoreMesh(
        core_axis_name="c", subcore_axis_name="s", num_cores=NC, num_subcores=NS
    )


def _gather_phase_a(H, Xp, src, dst):
    M = N_EDGES
    C = 200
    per_tile = M // NW
    iters = per_tile // C
    f32 = jnp.float32

    @functools.partial(
        pl.kernel,
        out_type=[
            jax.ShapeDtypeStruct((M, D_H), f32),
            jax.ShapeDtypeStruct((M, D_H), f32),
            jax.ShapeDtypeStruct((M, XW), f32),
            jax.ShapeDtypeStruct((M, XW), f32),
        ],
        mesh=_mesh(),
        compiler_params=pltpu.CompilerParams(use_tc_tiling_on_sc=False),
        scratch_types=[
            pltpu.VMEM((C,), jnp.int32),
            pltpu.VMEM((C,), jnp.int32),
            pltpu.VMEM((C, D_H), f32),
            pltpu.VMEM((C, D_H), f32),
            pltpu.VMEM((C, XW), f32),
            pltpu.VMEM((C, XW), f32),
            pltpu.SemaphoreType.DMA,
        ],
    )
    def k(h_hbm, x_hbm, src_hbm, dst_hbm, hd_o, hs_o, xs_o, xd_o,
          sbuf, dbuf, hdb, hsb, xsb, xdb, sem):
        c = lax.axis_index("c")
        s = lax.axis_index("s")
        wid = s * NC + c

        def body(kk, carry):
            base = wid * per_tile + kk * C
            pltpu.sync_copy(src_hbm.at[pl.ds(base, C)], sbuf)
            pltpu.sync_copy(dst_hbm.at[pl.ds(base, C)], dbuf)
            c1 = pltpu.async_copy(h_hbm.at[dbuf], hdb, sem)
            c2 = pltpu.async_copy(h_hbm.at[sbuf], hsb, sem)
            c3 = pltpu.async_copy(x_hbm.at[sbuf], xsb, sem)
            c4 = pltpu.async_copy(x_hbm.at[dbuf], xdb, sem)
            c1.wait()
            c2.wait()
            c3.wait()
            c4.wait()
            pltpu.sync_copy(hdb, hd_o.at[pl.ds(base, C)])
            pltpu.sync_copy(hsb, hs_o.at[pl.ds(base, C)])
            pltpu.sync_copy(xsb, xs_o.at[pl.ds(base, C)])
            pltpu.sync_copy(xdb, xd_o.at[pl.ds(base, C)])
            return carry

        lax.fori_loop(0, iters, body, 0)

    return k(H, Xp, src, dst)


def _gather_phase_b(q, KV, src, dst):
    M = N_EDGES
    C = 200
    per_tile = M // NW
    iters = per_tile // C
    f32 = jnp.float32

    @functools.partial(
        pl.kernel,
        out_type=[
            jax.ShapeDtypeStruct((M, D_H), f32),
            jax.ShapeDtypeStruct((M, 2 * D_H), f32),
        ],
        mesh=_mesh(),
        compiler_params=pltpu.CompilerParams(use_tc_tiling_on_sc=False),
        scratch_types=[
            pltpu.VMEM((C,), jnp.int32),
            pltpu.VMEM((C,), jnp.int32),
            pltpu.VMEM((C, D_H), f32),
            pltpu.VMEM((C, 2 * D_H), f32),
            pltpu.SemaphoreType.DMA,
        ],
    )
    def k(q_hbm, kv_hbm, src_hbm, dst_hbm, qd_o, kvs_o, sbuf, dbuf, qdb, kvb, sem):
        c = lax.axis_index("c")
        s = lax.axis_index("s")
        wid = s * NC + c

        def body(kk, carry):
            base = wid * per_tile + kk * C
            pltpu.sync_copy(src_hbm.at[pl.ds(base, C)], sbuf)
            pltpu.sync_copy(dst_hbm.at[pl.ds(base, C)], dbuf)
            c1 = pltpu.async_copy(q_hbm.at[dbuf], qdb, sem)
            c2 = pltpu.async_copy(kv_hbm.at[sbuf], kvb, sem)
            c1.wait()
            c2.wait()
            pltpu.sync_copy(qdb, qd_o.at[pl.ds(base, C)])
            pltpu.sync_copy(kvb, kvs_o.at[pl.ds(base, C)])
            return carry

        lax.fori_loop(0, iters, body, 0)

    return k(q, KV, src, dst)


def _gather_rows16(tbl, idx):
    M = N_EDGES
    C = 1000
    per_tile = M // NW
    iters = per_tile // C
    f32 = jnp.float32

    @functools.partial(
        pl.kernel,
        out_type=jax.ShapeDtypeStruct((M, XW), f32),
        mesh=_mesh(),
        compiler_params=pltpu.CompilerParams(use_tc_tiling_on_sc=False),
        scratch_types=[
            pltpu.VMEM((C,), jnp.int32),
            pltpu.VMEM((C, XW), f32),
            pltpu.SemaphoreType.DMA,
        ],
    )
    def k(t_hbm, idx_hbm, o_hbm, ibuf, rbuf, sem):
        c = lax.axis_index("c")
        s = lax.axis_index("s")
        wid = s * NC + c

        def body(kk, carry):
            base = wid * per_tile + kk * C
            pltpu.sync_copy(idx_hbm.at[pl.ds(base, C)], ibuf)
            pltpu.async_copy(t_hbm.at[ibuf], rbuf, sem).wait()
            pltpu.sync_copy(rbuf, o_hbm.at[pl.ds(base, C)])
            return carry

        lax.fori_loop(0, iters, body, 0)

    return k(tbl, idx)


def _scatter_add3(PS, dst, W, zblk):
    M = N_EDGES
    N = N_NODES
    C = 1000
    per_core = M // NC
    per_tile = per_core // NS
    iters = per_tile // C
    ZCH = 1000
    nz = N // ZCH
    zrounds = (nz + NS - 1) // NS
    f32 = jnp.float32
    NP = len(PS)

    @functools.partial(
        pl.kernel,
        out_type=[jax.ShapeDtypeStruct((NC * N, W), f32) for _ in range(NP)],
        mesh=_mesh(),
        compiler_params=pltpu.CompilerParams(use_tc_tiling_on_sc=False),
        scratch_types=[
            pltpu.VMEM((C, W), f32),
            pltpu.VMEM((C,), jnp.int32),
            pltpu.VMEM_SHARED((N, W), f32),
        ],
    )
    def k(*refs):
        p_hbms = refs[:NP]
        dst_hbm = refs[NP]
        z_hbm = refs[NP + 1]
        o_hbms = refs[NP + 2:NP + 2 + NP]
        pbuf, ibuf, acc = refs[NP + 2 + NP:]
        c = lax.axis_index("c")
        s = lax.axis_index("s")
        for p_hbm, o_hbm in zip(p_hbms, o_hbms):
            for g in range(zrounds):
                cid = g * NS + s

                @pl.when(cid < nz)
                def _():
                    pltpu.sync_copy(z_hbm, acc.at[pl.ds(cid * ZCH, ZCH)])

            plsc.subcore_barrier()

            def body(kk, carry):
                base = c * per_core + s * per_tile + kk * C
                pltpu.sync_copy(dst_hbm.at[pl.ds(base, C)], ibuf)
                pltpu.sync_copy(p_hbm.at[pl.ds(base, C)], pbuf)
                pltpu.sync_copy(pbuf, acc.at[ibuf], add=True)
                return carry

            lax.fori_loop(0, iters, body, 0)
            plsc.subcore_barrier()
            for g in range(zrounds):
                cid = g * NS + s

                @pl.when(cid < nz)
                def _():
                    pltpu.sync_copy(
                        acc.at[pl.ds(cid * ZCH, ZCH)],
                        o_hbm.at[pl.ds(c * N + cid * ZCH, ZCH)],
                    )

            plsc.subcore_barrier()

    out = k(*PS, dst, zblk)
    return list(out) if isinstance(out, (list, tuple)) else [out]


_REDGE = 4000
_RNODE = 2000


def _edge_mlp(Hd, Hs, Xs, Xd, E, Wm1, bm1, Wm2, bm2, Wx8, bx8, We, be):
    R = _REDGE
    G = N_EDGES // R
    f32 = jnp.float32

    def body(hd, hs, xs, xd, e, wm1, bm1r, wm2, bm2r, wx, bxr, we, ber,
             p0_o, p1_o, p2_o, e1_o):
        rel = xs[...] - xd[...]
        d2 = jnp.sum(rel * rel, axis=1, keepdims=True)
        feat = jnp.concatenate([hd[...], hs[...], d2, e[...]], axis=1)
        m = _silu(jnp.dot(feat, wm1[...], preferred_element_type=f32) + bm1r[...])
        m_ij = _silu(jnp.dot(m, wm2[...], preferred_element_type=f32) + bm2r[...])
        coef = jnp.tanh(jnp.dot(m_ij, wx[...], preferred_element_type=f32) + bxr[...])[:, 0:1]
        rc = jnp.concatenate(
            [rel[:, 0:3] * coef, jnp.ones((R, 1), f32), jnp.zeros((R, 4), f32)],
            axis=1,
        )
        p0_o[...] = m_ij[:, :24]
        p1_o[...] = m_ij[:, 24:48]
        p2_o[...] = jnp.concatenate([m_ij[:, 48:], rc], axis=1)
        e1_o[...] = e[...] + jnp.dot(
            jnp.concatenate([e[...], m_ij], axis=1), we[...],
            preferred_element_type=f32) + ber[...]

    row = lambda i: (i, 0)
    fix = lambda i: (0, 0)
    return pl.pallas_call(
        body,
        grid=(G,),
        in_specs=[
            pl.BlockSpec((R, D_H), row),
            pl.BlockSpec((R, D_H), row),
            pl.BlockSpec((R, XW), row),
            pl.BlockSpec((R, XW), row),
            pl.BlockSpec((R, D_E), row),
            pl.BlockSpec((2 * D_H + 1 + D_E, D_H), fix),
            pl.BlockSpec((1, D_H), fix),
            pl.BlockSpec((D_H, D_H), fix),
            pl.BlockSpec((1, D_H), fix),
            pl.BlockSpec((D_H, 8), fix),
            pl.BlockSpec((1, 8), fix),
            pl.BlockSpec((D_E + D_H, D_E), fix),
            pl.BlockSpec((1, D_E), fix),
        ],
        out_specs=[
            pl.BlockSpec((R, 24), row),
            pl.BlockSpec((R, 24), row),
            pl.BlockSpec((R, 24), row),
            pl.BlockSpec((R, D_E), row),
        ],
        out_shape=[
            jax.ShapeDtypeStruct((N_EDGES, 24), f32),
            jax.ShapeDtypeStruct((N_EDGES, 24), f32),
            jax.ShapeDtypeStruct((N_EDGES, 24), f32),
            jax.ShapeDtypeStruct((N_EDGES, D_E), f32),
        ],
    )(Hd, Hs, Xs, Xd, E, Wm1, bm1, Wm2, bm2, Wx8, bx8, We, be)


def _node_update(H, Xp, S0, S1, S2, Wh1, bh1, Wh2, bh2, Wq, Wk, Wv):
    R = _RNODE
    G = N_NODES // R
    f32 = jnp.float32

    def body(h, xp, s0a, s0b, s1a, s1b, s2a, s2b, wh1, bh1r, wh2, bh2r, wq, wk, wv,
             h1_o, x1_o, q_o, kv_o, inv_o):
        s0 = s0a[...] + s0b[...]
        s1 = s1a[...] + s1b[...]
        s2 = s2a[...] + s2b[...]
        agg = jnp.concatenate([s0, s1, s2[:, :16]], axis=1)
        h1 = h[...] + jnp.dot(
            _silu(jnp.dot(jnp.concatenate([h[...], agg], axis=1), wh1[...],
                          preferred_element_type=f32) + bh1r[...]),
            wh2[...], preferred_element_type=f32) + bh2r[...]
        deg = s2[:, 19:20]
        inv = 1.0 / (deg + 1.0)
        upd = jnp.concatenate([s2[:, 16:19] * inv, jnp.zeros((R, XW - 3), f32)], axis=1)
        h1_o[...] = h1
        x1_o[...] = xp[...] + upd
        q_o[...] = jnp.dot(h1, wq[...], preferred_element_type=f32)
        kv_o[...] = jnp.concatenate(
            [jnp.dot(h1, wk[...], preferred_element_type=f32),
             jnp.dot(h1, wv[...], preferred_element_type=f32)], axis=1)
        inv_o[...] = jnp.concatenate([inv, jnp.zeros((R, 7), f32)], axis=1)

    row = lambda i: (i, 0)
    fix = lambda i: (0, 0)
    s40row = lambda i: (i, 0)
    s40row_b = lambda i: (i, 0)
    return pl.pallas_call(
        body,
        grid=(G,),
        in_specs=[
            pl.BlockSpec((R, D_H), row),
            pl.BlockSpec((R, XW), row),
            pl.BlockSpec((R, 24), s40row),
            pl.BlockSpec((R, 24), s40row_b),
            pl.BlockSpec((R, 24), s40row),
            pl.BlockSpec((R, 24), s40row_b),
            pl.BlockSpec((R, 24), s40row),
            pl.BlockSpec((R, 24), s40row_b),
            pl.BlockSpec((2 * D_H, D_H), fix),
            pl.BlockSpec((1, D_H), fix),
            pl.BlockSpec((D_H, D_H), fix),
            pl.BlockSpec((1, D_H), fix),
            pl.BlockSpec((D_H, D_H), fix),
            pl.BlockSpec((D_H, D_H), fix),
            pl.BlockSpec((D_H, D_H), fix),
        ],
        out_specs=[
            pl.BlockSpec((R, D_H), row),
            pl.BlockSpec((R, XW), row),
            pl.BlockSpec((R, D_H), row),
            pl.BlockSpec((R, 2 * D_H), row),
            pl.BlockSpec((R, 8), row),
        ],
        out_shape=[
            jax.ShapeDtypeStruct((N_NODES, D_H), f32),
            jax.ShapeDtypeStruct((N_NODES, XW), f32),
            jax.ShapeDtypeStruct((N_NODES, D_H), f32),
            jax.ShapeDtypeStruct((N_NODES, 2 * D_H), f32),
            jax.ShapeDtypeStruct((N_NODES, 8), f32),
        ],
    )(H, Xp, S0[:N_NODES], S0[N_NODES:], S1[:N_NODES], S1[N_NODES:],
      S2[:N_NODES], S2[N_NODES:], Wh1, bh1, Wh2, bh2, Wq, Wk, Wv)


def _logits(qd, KVs, E1, Shead, Web8):
    R = _REDGE
    G = N_EDGES // R
    f32 = jnp.float32

    def body(q, kv, e1, sh, web, l_o, g_o):
        i = pl.program_id(0)
        prod = q[...] * kv[:, :D_H]
        l = (jnp.dot(prod, sh[...], preferred_element_type=f32) * (1.0 / np.sqrt(D_HEAD))
             + jnp.dot(e1[...], web[...], preferred_element_type=f32))
        l_o[...] = l
        bm = jnp.max(l, axis=0)

        @pl.when(i == 0)
        def _():
            g_o[...] = jnp.full((8, 128), -1e30, f32)

        g_o[...] = jnp.maximum(g_o[...], jnp.broadcast_to(bm[:, None], (8, 128)))

    row = lambda i: (i, 0)
    fix = lambda i: (0, 0)
    return pl.pallas_call(
        body,
        grid=(G,),
        in_specs=[
            pl.BlockSpec((R, D_H), row),
            pl.BlockSpec((R, 2 * D_H), row),
            pl.BlockSpec((R, D_E), row),
            pl.BlockSpec((D_H, 8), fix),
            pl.BlockSpec((D_E, 8), fix),
        ],
        out_specs=[pl.BlockSpec((R, 8), row), pl.BlockSpec((8, 128), fix)],
        out_shape=[
            jax.ShapeDtypeStruct((N_EDGES, 8), f32),
            jax.ShapeDtypeStruct((8, 128), f32),
        ],
    )(qd, KVs, E1, Shead, Web8)


def _alpha(logits, gmax):
    R = _REDGE
    G = N_EDGES // R
    f32 = jnp.float32

    def body(l, g, a_o):
        gv = g[:, 0:1]
        a = jnp.exp(l[...] - gv[:, 0][None, :])
        mask = lax.broadcasted_iota(jnp.int32, (R, XW), 1) < HEADS
        a_o[...] = jnp.where(mask, jnp.concatenate([a, jnp.zeros((R, 8), f32)], axis=1), 0.0)

    row = lambda i: (i, 0)
    fix = lambda i: (0, 0)
    return pl.pallas_call(
        body,
        grid=(G,),
        in_specs=[pl.BlockSpec((R, 8), row), pl.BlockSpec((8, 128), fix)],
        out_specs=pl.BlockSpec((R, XW), row),
        out_shape=jax.ShapeDtypeStruct((N_EDGES, XW), f32),
    )(logits, gmax)


def _denr(D16):
    R = _RNODE
    G = N_NODES // R
    f32 = jnp.float32

    def body(a, b, o):
        o[...] = 1.0 / (a[...] + b[...] + 1e-9)

    return pl.pallas_call(
        body,
        grid=(G,),
        in_specs=[
            pl.BlockSpec((R, XW), lambda i: (i, 0)),
            pl.BlockSpec((R, XW), lambda i: (i, 0)),
        ],
        out_specs=pl.BlockSpec((R, XW), lambda i: (i, 0)),
        out_shape=jax.ShapeDtypeStruct((N_NODES, XW), f32),
    )(D16[:N_NODES], D16[N_NODES:])


def _attn_msgs(alpha16, denrd, KVs, Xs, Xd, E1, Ehead, We2, be2):
    R = _REDGE
    G = N_EDGES // R
    f32 = jnp.float32

    def body(a, dr, kv, xs, xd, e1, eh, we2, be2r, p0_o, p1_o, p2_o, e2_o):
        attn = a[:, :HEADS] * dr[:, :HEADS]
        xw = jnp.mean(attn, axis=1, keepdims=True)
        rel = xs[...] - xd[...]
        relxw = jnp.concatenate([rel[:, 0:3] * xw, jnp.zeros((R, 5), f32)], axis=1)
        av = kv[:, D_H:] * jnp.dot(attn, eh[...], preferred_element_type=f32)
        p0_o[...] = av[:, :24]
        p1_o[...] = av[:, 24:48]
        p2_o[...] = jnp.concatenate([av[:, 48:], relxw], axis=1)
        e2_o[...] = e1[...] + jnp.dot(
            jnp.concatenate([e1[...], attn], axis=1), we2[...],
            preferred_element_type=f32) + be2r[...]

    row = lambda i: (i, 0)
    fix = lambda i: (0, 0)
    return pl.pallas_call(
        body,
        grid=(G,),
        in_specs=[
            pl.BlockSpec((R, XW), row),
            pl.BlockSpec((R, XW), row),
            pl.BlockSpec((R, 2 * D_H), row),
            pl.BlockSpec((R, XW), row),
            pl.BlockSpec((R, XW), row),
            pl.BlockSpec((R, D_E), row),
            pl.BlockSpec((HEADS, D_H), fix),
            pl.BlockSpec((D_E + HEADS, D_E), fix),
            pl.BlockSpec((1, D_E), fix),
        ],
        out_specs=[
            pl.BlockSpec((R, 24), row),
            pl.BlockSpec((R, 24), row),
            pl.BlockSpec((R, 24), row),
            pl.BlockSpec((R, D_E), row),
        ],
        out_shape=[
            jax.ShapeDtypeStruct((N_EDGES, 24), f32),
            jax.ShapeDtypeStruct((N_EDGES, 24), f32),
            jax.ShapeDtypeStruct((N_EDGES, 24), f32),
            jax.ShapeDtypeStruct((N_EDGES, D_E), f32),
        ],
    )(alpha16, denrd, KVs, Xs, Xd, E1, Ehead, We2, be2)


def _final_nodes(H1, T0, T1, T2, inv8, X1, Z, batch3, Wo, Wz, bz):
    R = _RNODE
    G = N_NODES // R
    f32 = jnp.float32

    def body(h1, s0a, s0b, s1a, s1b, s2a, s2b, inv, x1, z, b3, wo, wz, bzr,
             h2_o, z1_o, x2_o, gs_o):
        i = pl.program_id(0)
        s0 = s0a[...] + s0b[...]
        s1 = s1a[...] + s1b[...]
        s2 = s2a[...] + s2b[...]
        agg2 = jnp.concatenate([s0, s1, s2[:, :16]], axis=1)
        h2 = h1[...] + jnp.dot(agg2, wo[...], preferred_element_type=f32)
        h2_o[...] = h2
        z1_o[...] = z[...] + _silu(jnp.dot(h2, wz[...], preferred_element_type=f32) + bzr[...])
        upd = jnp.concatenate(
            [s2[:, 16:19] * inv[:, 0:1], jnp.zeros((R, XW - 3), f32)], axis=1)
        x2 = x1[...] + upd
        x2_o[...] = x2
        lanes = lax.broadcasted_iota(jnp.int32, (R, XW), 1)
        x2c = jnp.where(lanes == 3, 1.0, x2)
        b = b3[0, 0, :]
        gids = lax.broadcasted_iota(jnp.int32, (R, N_GRAPHS), 1)
        onehot = (jnp.broadcast_to(b[:, None], (R, N_GRAPHS)) == gids).astype(f32)
        blk = lax.dot_general(onehot, x2c, (((0,), (0,)), ((), ())),
                              preferred_element_type=f32)

        @pl.when(i == 0)
        def _():
            gs_o[...] = jnp.zeros((N_GRAPHS, XW), f32)

        gs_o[...] = gs_o[...] + blk

    row = lambda i: (i, 0)
    fix = lambda i: (0, 0)
    return pl.pallas_call(
        body,
        grid=(G,),
        in_specs=[
            pl.BlockSpec((R, D_H), row),
            pl.BlockSpec((R, 24), row),
            pl.BlockSpec((R, 24), row),
            pl.BlockSpec((R, 24), row),
            pl.BlockSpec((R, 24), row),
            pl.BlockSpec((R, 24), row),
            pl.BlockSpec((R, 24), row),
            pl.BlockSpec((R, 8), row),
            pl.BlockSpec((R, XW), row),
            pl.BlockSpec((R, D_H), row),
            pl.BlockSpec((1, 1, R), lambda i: (i, 0, 0)),
            pl.BlockSpec((D_H, D_H), fix),
            pl.BlockSpec((D_H, D_H), fix),
            pl.BlockSpec((1, D_H), fix),
        ],
        out_specs=[
            pl.BlockSpec((R, D_H), row),
            pl.BlockSpec((R, D_H), row),
            pl.BlockSpec((R, XW), row),
            pl.BlockSpec((N_GRAPHS, XW), fix),
        ],
        out_shape=[
            jax.ShapeDtypeStruct((N_NODES, D_H), f32),
            jax.ShapeDtypeStruct((N_NODES, D_H), f32),
            jax.ShapeDtypeStruct((N_NODES, XW), f32),
            jax.ShapeDtypeStruct((N_GRAPHS, XW), f32),
        ],
    )(H1, T0[:N_NODES], T0[N_NODES:], T1[:N_NODES], T1[N_NODES:],
      T2[:N_NODES], T2[N_NODES:], inv8, X1, Z, batch3, Wo, Wz, bz)


def _center(X2, batch3, gsums):
    R = _RNODE
    G = N_NODES // R
    f32 = jnp.float32

    def body(x2, b3, gs, o):
        mean = gs[...] * (1.0 / (gs[:, 3:4] + 1e-9))
        b = b3[0, 0, :]
        gids = lax.broadcasted_iota(jnp.int32, (R, N_GRAPHS), 1)
        onehot = (jnp.broadcast_to(b[:, None], (R, N_GRAPHS)) == gids).astype(f32)
        o[...] = x2[...] - jnp.dot(onehot, mean, preferred_element_type=f32)

    return pl.pallas_call(
        body,
        grid=(G,),
        in_specs=[
            pl.BlockSpec((R, XW), lambda i: (i, 0)),
            pl.BlockSpec((1, 1, R), lambda i: (i, 0, 0)),
            pl.BlockSpec((N_GRAPHS, XW), lambda i: (0, 0)),
        ],
        out_specs=pl.BlockSpec((R, XW), lambda i: (i, 0)),
        out_shape=jax.ShapeDtypeStruct((N_NODES, XW), f32),
    )(X2, batch3, gsums)


def kernel(batch, X, H, E_idx, E, Z, params):
    p = params
    f32 = jnp.float32
    src = E_idx[0].astype(jnp.int32)
    dst = E_idx[1].astype(jnp.int32)
    Xp = jnp.pad(X.astype(f32), ((0, 0), (0, XW - 3)))
    batch3 = batch.astype(jnp.int32).reshape(N_NODES // _RNODE, 1, _RNODE)

    z24 = jnp.zeros((1000, 24), f32)
    z16 = jnp.zeros((1000, XW), f32)

    bm1 = p['b_m1'].reshape(1, D_H)
    bm2 = p['b_m2'].reshape(1, D_H)
    Wx8 = jnp.pad(p['W_x'], ((0, 0), (0, 7)))
    bx8 = jnp.pad(p['b_x'].reshape(1, 1), ((0, 0), (0, 7)))
    be = p['b_e'].reshape(1, D_E)
    bh1 = p['b_h1'].reshape(1, D_H)
    bh2 = p['b_h2'].reshape(1, D_H)
    be2 = p['b_e2'].reshape(1, D_E)
    bz = p['b_z'].reshape(1, D_H)
    Web8 = jnp.pad(p['W_eb'], ((0, 0), (0, 8 - HEADS)))
    shead = np.zeros((D_H, 8), np.float32)
    for h in range(HEADS):
        shead[h * D_HEAD:(h + 1) * D_HEAD, h] = 1.0
    Shead = jnp.asarray(shead)
    ehead = np.zeros((HEADS, D_H), np.float32)
    for h in range(HEADS):
        ehead[h, h * D_HEAD:(h + 1) * D_HEAD] = 1.0
    Ehead = jnp.asarray(ehead)

    Hd, Hs, Xs, Xd = _gather_phase_a(H, Xp, src, dst)
    P0, P1, P2, E1 = _edge_mlp(Hd, Hs, Xs, Xd, E, p['W_m1'], bm1, p['W_m2'], bm2,
                               Wx8, bx8, p['W_e'], be)
    S0, S1, S2 = _scatter_add3([P0, P1, P2], dst, 24, z24)
    H1, X1, q, KV, inv8 = _node_update(H, Xp, S0, S1, S2, p['W_h1'], bh1,
                                       p['W_h2'], bh2, p['W_q'], p['W_k'], p['W_v'])
    qd, KVs = _gather_phase_b(q, KV, src, dst)
    logits, gmax = _logits(qd, KVs, E1, Shead, Web8)
    alpha16 = _alpha(logits, gmax)
    D16, = _scatter_add3([alpha16], dst, XW, z16)
    denr = _denr(D16)
    denrd = _gather_rows16(denr, dst)
    Q0, Q1, Q2, E2 = _attn_msgs(alpha16, denrd, KVs, Xs, Xd, E1, Ehead,
                                p['W_e2'], be2)
    T0, T1, T2 = _scatter_add3([Q0, Q1, Q2], dst, 24, z24)
    H2, Z1, X2, gsums = _final_nodes(H1, T0, T1, T2, inv8, X1, Z, batch3,
                                     p['W_o'], p['W_z'], bz)
    Xout = _center(X2, batch3, gsums)
    return (Xout[:, :3], H2, E2, Z1)


# speedup vs baseline: 11.7450x; 1.0147x over previous
import functools

import jax
import jax.numpy as jnp
import numpy as np
from jax import lax
from jax.experimental import pallas as pl
from jax.experimental.pallas import tpu as pltpu
from jax.experimental.pallas import tpu_sc as plsc

N_NODES = 50000
N_EDGES = 800000
D_H = 64
D_E = 32
HEADS = 4
D_HEAD = D_H // HEADS
N_GRAPHS = 64
XW = 16
NC, NS = 2, 16
NW = NC * NS


def _silu(x):
    return x * jax.nn.sigmoid(x)


def _mesh():
    return plsc.VectorSubcoreMesh(
        core_axis_name="c", subcore_axis_name="s", num_cores=NC, num_subcores=NS
    )


def _gather_phase_a(H, Xp, src, dst):
    M = N_EDGES
    C = 200
    per_tile = M // NW
    iters = per_tile // C
    f32 = jnp.float32

    @functools.partial(
        pl.kernel,
        out_type=[
            jax.ShapeDtypeStruct((M, D_H), f32),
            jax.ShapeDtypeStruct((M, D_H), f32),
            jax.ShapeDtypeStruct((M, XW), f32),
            jax.ShapeDtypeStruct((M, XW), f32),
        ],
        mesh=_mesh(),
        compiler_params=pltpu.CompilerParams(use_tc_tiling_on_sc=False),
        scratch_types=[
            pltpu.VMEM((C,), jnp.int32),
            pltpu.VMEM((C,), jnp.int32),
            pltpu.VMEM((C, D_H), f32),
            pltpu.VMEM((C, D_H), f32),
            pltpu.VMEM((C, XW), f32),
            pltpu.VMEM((C, XW), f32),
            pltpu.SemaphoreType.DMA,
        ],
    )
    def k(h_hbm, x_hbm, src_hbm, dst_hbm, hd_o, hs_o, xs_o, xd_o,
          sbuf, dbuf, hdb, hsb, xsb, xdb, sem):
        c = lax.axis_index("c")
        s = lax.axis_index("s")
        wid = s * NC + c

        def body(kk, carry):
            base = wid * per_tile + kk * C
            pltpu.sync_copy(src_hbm.at[pl.ds(base, C)], sbuf)
            pltpu.sync_copy(dst_hbm.at[pl.ds(base, C)], dbuf)
            c1 = pltpu.async_copy(h_hbm.at[dbuf], hdb, sem)
            c2 = pltpu.async_copy(h_hbm.at[sbuf], hsb, sem)
            c3 = pltpu.async_copy(x_hbm.at[sbuf], xsb, sem)
            c4 = pltpu.async_copy(x_hbm.at[dbuf], xdb, sem)
            c1.wait()
            c2.wait()
            c3.wait()
            c4.wait()
            pltpu.sync_copy(hdb, hd_o.at[pl.ds(base, C)])
            pltpu.sync_copy(hsb, hs_o.at[pl.ds(base, C)])
            pltpu.sync_copy(xsb, xs_o.at[pl.ds(base, C)])
            pltpu.sync_copy(xdb, xd_o.at[pl.ds(base, C)])
            return carry

        lax.fori_loop(0, iters, body, 0)

    return k(H, Xp, src, dst)


def _gather_phase_b(q, KV, src, dst):
    M = N_EDGES
    C = 200
    per_tile = M // NW
    iters = per_tile // C
    f32 = jnp.float32

    @functools.partial(
        pl.kernel,
        out_type=[
            jax.ShapeDtypeStruct((M, D_H), f32),
            jax.ShapeDtypeStruct((M, 2 * D_H), f32),
        ],
        mesh=_mesh(),
        compiler_params=pltpu.CompilerParams(use_tc_tiling_on_sc=False),
        scratch_types=[
            pltpu.VMEM((C,), jnp.int32),
            pltpu.VMEM((C,), jnp.int32),
            pltpu.VMEM((C, D_H), f32),
            pltpu.VMEM((C, 2 * D_H), f32),
            pltpu.SemaphoreType.DMA,
        ],
    )
    def k(q_hbm, kv_hbm, src_hbm, dst_hbm, qd_o, kvs_o, sbuf, dbuf, qdb, kvb, sem):
        c = lax.axis_index("c")
        s = lax.axis_index("s")
        wid = s * NC + c

        def body(kk, carry):
            base = wid * per_tile + kk * C
            pltpu.sync_copy(src_hbm.at[pl.ds(base, C)], sbuf)
            pltpu.sync_copy(dst_hbm.at[pl.ds(base, C)], dbuf)
            c1 = pltpu.async_copy(q_hbm.at[dbuf], qdb, sem)
            c2 = pltpu.async_copy(kv_hbm.at[sbuf], kvb, sem)
            c1.wait()
            c2.wait()
            pltpu.sync_copy(qdb, qd_o.at[pl.ds(base, C)])
            pltpu.sync_copy(kvb, kvs_o.at[pl.ds(base, C)])
            return carry

        lax.fori_loop(0, iters, body, 0)

    return k(q, KV, src, dst)


def _gather_rows16(tbl, idx):
    M = N_EDGES
    C = 1000
    per_tile = M // NW
    iters = per_tile // C
    f32 = jnp.float32

    @functools.partial(
        pl.kernel,
        out_type=jax.ShapeDtypeStruct((M, XW), f32),
        mesh=_mesh(),
        compiler_params=pltpu.CompilerParams(use_tc_tiling_on_sc=False),
        scratch_types=[
            pltpu.VMEM((C,), jnp.int32),
            pltpu.VMEM((C, XW), f32),
            pltpu.SemaphoreType.DMA,
        ],
    )
    def k(t_hbm, idx_hbm, o_hbm, ibuf, rbuf, sem):
        c = lax.axis_index("c")
        s = lax.axis_index("s")
        wid = s * NC + c

        def body(kk, carry):
            base = wid * per_tile + kk * C
            pltpu.sync_copy(idx_hbm.at[pl.ds(base, C)], ibuf)
            pltpu.async_copy(t_hbm.at[ibuf], rbuf, sem).wait()
            pltpu.sync_copy(rbuf, o_hbm.at[pl.ds(base, C)])
            return carry

        lax.fori_loop(0, iters, body, 0)

    return k(tbl, idx)


def _scatter_add3(PS, dst, W, zblk):
    M = N_EDGES
    N = N_NODES
    C = 1000
    per_core = M // NC
    per_tile = per_core // NS
    iters = per_tile // C
    ZCH = 1000
    nz = N // ZCH
    zrounds = (nz + NS - 1) // NS
    f32 = jnp.float32
    NP = len(PS)

    @functools.partial(
        pl.kernel,
        out_type=[jax.ShapeDtypeStruct((NC * N, W), f32) for _ in range(NP)],
        mesh=_mesh(),
        compiler_params=pltpu.CompilerParams(use_tc_tiling_on_sc=False),
        scratch_types=[
            pltpu.VMEM((C, W), f32),
            pltpu.VMEM((C,), jnp.int32),
            pltpu.VMEM_SHARED((N, W), f32),
        ],
    )
    def k(*refs):
        p_hbms = refs[:NP]
        dst_hbm = refs[NP]
        z_hbm = refs[NP + 1]
        o_hbms = refs[NP + 2:NP + 2 + NP]
        pbuf, ibuf, acc = refs[NP + 2 + NP:]
        c = lax.axis_index("c")
        s = lax.axis_index("s")
        for p_hbm, o_hbm in zip(p_hbms, o_hbms):
            for g in range(zrounds):
                cid = g * NS + s

                @pl.when(cid < nz)
                def _():
                    pltpu.sync_copy(z_hbm, acc.at[pl.ds(cid * ZCH, ZCH)])

            plsc.subcore_barrier()

            def body(kk, carry):
                base = c * per_core + s * per_tile + kk * C
                pltpu.sync_copy(dst_hbm.at[pl.ds(base, C)], ibuf)
                pltpu.sync_copy(p_hbm.at[pl.ds(base, C)], pbuf)
                pltpu.sync_copy(pbuf, acc.at[ibuf], add=True)
                return carry

            lax.fori_loop(0, iters, body, 0)
            plsc.subcore_barrier()
            for g in range(zrounds):
                cid = g * NS + s

                @pl.when(cid < nz)
                def _():
                    pltpu.sync_copy(
                        acc.at[pl.ds(cid * ZCH, ZCH)],
                        o_hbm.at[pl.ds(c * N + cid * ZCH, ZCH)],
                    )

            plsc.subcore_barrier()

    out = k(*PS, dst, zblk)
    return list(out) if isinstance(out, (list, tuple)) else [out]


_REDGE = 4000
_RNODE = 2000


def _edge_mlp(Hd, Hs, Xs, Xd, E, Wm1, bm1, Wm2, bm2, Wx8, bx8, We, be):
    R = _REDGE
    G = N_EDGES // R
    f32 = jnp.float32

    def body(hd, hs, xs, xd, e, wm1, bm1r, wm2, bm2r, wx, bxr, we, ber,
             p0_o, p1_o, p2_o, e1_o):
        rel = xs[...] - xd[...]
        d2 = jnp.sum(rel * rel, axis=1, keepdims=True)
        feat = jnp.concatenate([hd[...], hs[...], d2, e[...]], axis=1)
        m = _silu(jnp.dot(feat, wm1[...], preferred_element_type=f32) + bm1r[...])
        m_ij = _silu(jnp.dot(m, wm2[...], preferred_element_type=f32) + bm2r[...])
        coef = jnp.tanh(jnp.dot(m_ij, wx[...], preferred_element_type=f32) + bxr[...])[:, 0:1]
        rc = jnp.concatenate(
            [rel[:, 0:3] * coef, jnp.ones((R, 1), f32), jnp.zeros((R, 4), f32)],
            axis=1,
        )
        p0_o[...] = m_ij[:, :24]
        p1_o[...] = m_ij[:, 24:48]
        p2_o[...] = jnp.concatenate([m_ij[:, 48:], rc], axis=1)
        e1_o[...] = e[...] + jnp.dot(
            jnp.concatenate([e[...], m_ij], axis=1), we[...],
            preferred_element_type=f32) + ber[...]

    row = lambda i: (i, 0)
    fix = lambda i: (0, 0)
    return pl.pallas_call(
        body,
        grid=(G,),
        in_specs=[
            pl.BlockSpec((R, D_H), row),
            pl.BlockSpec((R, D_H), row),
            pl.BlockSpec((R, XW), row),
            pl.BlockSpec((R, XW), row),
            pl.BlockSpec((R, D_E), row),
            pl.BlockSpec((2 * D_H + 1 + D_E, D_H), fix),
            pl.BlockSpec((1, D_H), fix),
            pl.BlockSpec((D_H, D_H), fix),
            pl.BlockSpec((1, D_H), fix),
            pl.BlockSpec((D_H, 8), fix),
            pl.BlockSpec((1, 8), fix),
            pl.BlockSpec((D_E + D_H, D_E), fix),
            pl.BlockSpec((1, D_E), fix),
        ],
        out_specs=[
            pl.BlockSpec((R, 24), row),
            pl.BlockSpec((R, 24), row),
            pl.BlockSpec((R, 24), row),
            pl.BlockSpec((R, D_E), row),
        ],
        out_shape=[
            jax.ShapeDtypeStruct((N_EDGES, 24), f32),
            jax.ShapeDtypeStruct((N_EDGES, 24), f32),
            jax.ShapeDtypeStruct((N_EDGES, 24), f32),
            jax.ShapeDtypeStruct((N_EDGES, D_E), f32),
        ],
    )(Hd, Hs, Xs, Xd, E, Wm1, bm1, Wm2, bm2, Wx8, bx8, We, be)


def _node_update(H, Xp, S0, S1, S2, Wh1, bh1, Wh2, bh2, Wq, Wk, Wv):
    R = _RNODE
    G = N_NODES // R
    f32 = jnp.float32

    def body(h, xp, s0a, s0b, s1a, s1b, s2a, s2b, wh1, bh1r, wh2, bh2r, wq, wk, wv,
             h1_o, x1_o, q_o, kv_o, inv_o):
        s0 = s0a[...] + s0b[...]
        s1 = s1a[...] + s1b[...]
        s2 = s2a[...] + s2b[...]
        agg = jnp.concatenate([s0, s1, s2[:, :16]], axis=1)
        h1 = h[...] + jnp.dot(
            _silu(jnp.dot(jnp.concatenate([h[...], agg], axis=1), wh1[...],
                          preferred_element_type=f32) + bh1r[...]),
            wh2[...], preferred_element_type=f32) + bh2r[...]
        deg = s2[:, 19:20]
        inv = 1.0 / (deg + 1.0)
        upd = jnp.concatenate([s2[:, 16:19] * inv, jnp.zeros((R, XW - 3), f32)], axis=1)
        h1_o[...] = h1
        x1_o[...] = xp[...] + upd
        q_o[...] = jnp.dot(h1, wq[...], preferred_element_type=f32)
        kv_o[...] = jnp.concatenate(
            [jnp.dot(h1, wk[...], preferred_element_type=f32),
             jnp.dot(h1, wv[...], preferred_element_type=f32)], axis=1)
        inv_o[...] = jnp.concatenate([inv, jnp.zeros((R, 7), f32)], axis=1)

    row = lambda i: (i, 0)
    fix = lambda i: (0, 0)
    s40row = lambda i: (i, 0)
    s40row_b = lambda i: (G + i, 0)
    return pl.pallas_call(
        body,
        grid=(G,),
        in_specs=[
            pl.BlockSpec((R, D_H), row),
            pl.BlockSpec((R, XW), row),
            pl.BlockSpec((R, 24), s40row),
            pl.BlockSpec((R, 24), s40row_b),
            pl.BlockSpec((R, 24), s40row),
            pl.BlockSpec((R, 24), s40row_b),
            pl.BlockSpec((R, 24), s40row),
            pl.BlockSpec((R, 24), s40row_b),
            pl.BlockSpec((2 * D_H, D_H), fix),
            pl.BlockSpec((1, D_H), fix),
            pl.BlockSpec((D_H, D_H), fix),
            pl.BlockSpec((1, D_H), fix),
            pl.BlockSpec((D_H, D_H), fix),
            pl.BlockSpec((D_H, D_H), fix),
            pl.BlockSpec((D_H, D_H), fix),
        ],
        out_specs=[
            pl.BlockSpec((R, D_H), row),
            pl.BlockSpec((R, XW), row),
            pl.BlockSpec((R, D_H), row),
            pl.BlockSpec((R, 2 * D_H), row),
            pl.BlockSpec((R, 8), row),
        ],
        out_shape=[
            jax.ShapeDtypeStruct((N_NODES, D_H), f32),
            jax.ShapeDtypeStruct((N_NODES, XW), f32),
            jax.ShapeDtypeStruct((N_NODES, D_H), f32),
            jax.ShapeDtypeStruct((N_NODES, 2 * D_H), f32),
            jax.ShapeDtypeStruct((N_NODES, 8), f32),
        ],
    )(H, Xp, S0, S0, S1, S1, S2, S2, Wh1, bh1, Wh2, bh2, Wq, Wk, Wv)


def _logits(qd, KVs, E1, Shead, Web8):
    R = _REDGE
    G = N_EDGES // R
    f32 = jnp.float32

    def body(q, kv, e1, sh, web, l_o, g_o):
        i = pl.program_id(0)
        prod = q[...] * kv[:, :D_H]
        l = (jnp.dot(prod, sh[...], preferred_element_type=f32) * (1.0 / np.sqrt(D_HEAD))
             + jnp.dot(e1[...], web[...], preferred_element_type=f32))
        l_o[...] = l
        bm = jnp.max(l, axis=0)

        @pl.when(i == 0)
        def _():
            g_o[...] = jnp.full((8, 128), -1e30, f32)

        g_o[...] = jnp.maximum(g_o[...], jnp.broadcast_to(bm[:, None], (8, 128)))

    row = lambda i: (i, 0)
    fix = lambda i: (0, 0)
    return pl.pallas_call(
        body,
        grid=(G,),
        in_specs=[
            pl.BlockSpec((R, D_H), row),
            pl.BlockSpec((R, 2 * D_H), row),
            pl.BlockSpec((R, D_E), row),
            pl.BlockSpec((D_H, 8), fix),
            pl.BlockSpec((D_E, 8), fix),
        ],
        out_specs=[pl.BlockSpec((R, 8), row), pl.BlockSpec((8, 128), fix)],
        out_shape=[
            jax.ShapeDtypeStruct((N_EDGES, 8), f32),
            jax.ShapeDtypeStruct((8, 128), f32),
        ],
    )(qd, KVs, E1, Shead, Web8)


def _alpha(logits, gmax):
    R = _REDGE
    G = N_EDGES // R
    f32 = jnp.float32

    def body(l, g, a_o):
        gv = g[:, 0:1]
        a = jnp.exp(l[...] - gv[:, 0][None, :])
        mask = lax.broadcasted_iota(jnp.int32, (R, XW), 1) < HEADS
        a_o[...] = jnp.where(mask, jnp.concatenate([a, jnp.zeros((R, 8), f32)], axis=1), 0.0)

    row = lambda i: (i, 0)
    fix = lambda i: (0, 0)
    return pl.pallas_call(
        body,
        grid=(G,),
        in_specs=[pl.BlockSpec((R, 8), row), pl.BlockSpec((8, 128), fix)],
        out_specs=pl.BlockSpec((R, XW), row),
        out_shape=jax.ShapeDtypeStruct((N_EDGES, XW), f32),
    )(logits, gmax)


def _denr(D16):
    R = _RNODE
    G = N_NODES // R
    f32 = jnp.float32

    def body(a, b, o):
        o[...] = 1.0 / (a[...] + b[...] + 1e-9)

    return pl.pallas_call(
        body,
        grid=(G,),
        in_specs=[
            pl.BlockSpec((R, XW), lambda i: (i, 0)),
            pl.BlockSpec((R, XW), lambda i: (G + i, 0)),
        ],
        out_specs=pl.BlockSpec((R, XW), lambda i: (i, 0)),
        out_shape=jax.ShapeDtypeStruct((N_NODES, XW), f32),
    )(D16, D16)


def _attn_msgs(alpha16, denrd, KVs, Xs, Xd, E1, Ehead, We2, be2):
    R = _REDGE
    G = N_EDGES // R
    f32 = jnp.float32

    def body(a, dr, kv, xs, xd, e1, eh, we2, be2r, p0_o, p1_o, p2_o, e2_o):
        attn = a[:, :HEADS] * dr[:, :HEADS]
        xw = jnp.mean(attn, axis=1, keepdims=True)
        rel = xs[...] - xd[...]
        relxw = jnp.concatenate([rel[:, 0:3] * xw, jnp.zeros((R, 5), f32)], axis=1)
        av = kv[:, D_H:] * jnp.dot(attn, eh[...], preferred_element_type=f32)
        p0_o[...] = av[:, :24]
        p1_o[...] = av[:, 24:48]
        p2_o[...] = jnp.concatenate([av[:, 48:], relxw], axis=1)
        e2_o[...] = e1[...] + jnp.dot(
            jnp.concatenate([e1[...], attn], axis=1), we2[...],
            preferred_element_type=f32) + be2r[...]

    row = lambda i: (i, 0)
    fix = lambda i: (0, 0)
    return pl.pallas_call(
        body,
        grid=(G,),
        in_specs=[
            pl.BlockSpec((R, XW), row),
            pl.BlockSpec((R, XW), row),
            pl.BlockSpec((R, 2 * D_H), row),
            pl.BlockSpec((R, XW), row),
            pl.BlockSpec((R, XW), row),
            pl.BlockSpec((R, D_E), row),
            pl.BlockSpec((HEADS, D_H), fix),
            pl.BlockSpec((D_E + HEADS, D_E), fix),
            pl.BlockSpec((1, D_E), fix),
        ],
        out_specs=[
            pl.BlockSpec((R, 24), row),
            pl.BlockSpec((R, 24), row),
            pl.BlockSpec((R, 24), row),
            pl.BlockSpec((R, D_E), row),
        ],
        out_shape=[
            jax.ShapeDtypeStruct((N_EDGES, 24), f32),
            jax.ShapeDtypeStruct((N_EDGES, 24), f32),
            jax.ShapeDtypeStruct((N_EDGES, 24), f32),
            jax.ShapeDtypeStruct((N_EDGES, D_E), f32),
        ],
    )(alpha16, denrd, KVs, Xs, Xd, E1, Ehead, We2, be2)


def _final_nodes(H1, T0, T1, T2, inv8, X1, Z, batch3, Wo, Wz, bz):
    R = _RNODE
    G = N_NODES // R
    f32 = jnp.float32

    def body(h1, s0a, s0b, s1a, s1b, s2a, s2b, inv, x1, z, b3, wo, wz, bzr,
             h2_o, z1_o, x2_o, gs_o):
        i = pl.program_id(0)
        s0 = s0a[...] + s0b[...]
        s1 = s1a[...] + s1b[...]
        s2 = s2a[...] + s2b[...]
        agg2 = jnp.concatenate([s0, s1, s2[:, :16]], axis=1)
        h2 = h1[...] + jnp.dot(agg2, wo[...], preferred_element_type=f32)
        h2_o[...] = h2
        z1_o[...] = z[...] + _silu(jnp.dot(h2, wz[...], preferred_element_type=f32) + bzr[...])
        upd = jnp.concatenate(
            [s2[:, 16:19] * inv[:, 0:1], jnp.zeros((R, XW - 3), f32)], axis=1)
        x2 = x1[...] + upd
        x2_o[...] = x2
        lanes = lax.broadcasted_iota(jnp.int32, (R, XW), 1)
        x2c = jnp.where(lanes == 3, 1.0, x2)
        b = b3[0, 0, :]
        gids = lax.broadcasted_iota(jnp.int32, (R, N_GRAPHS), 1)
        onehot = (jnp.broadcast_to(b[:, None], (R, N_GRAPHS)) == gids).astype(f32)
        blk = lax.dot_general(onehot, x2c, (((0,), (0,)), ((), ())),
                              preferred_element_type=f32)

        @pl.when(i == 0)
        def _():
            gs_o[...] = jnp.zeros((N_GRAPHS, XW), f32)

        gs_o[...] = gs_o[...] + blk

    row = lambda i: (i, 0)
    fix = lambda i: (0, 0)
    return pl.pallas_call(
        body,
        grid=(G,),
        in_specs=[
            pl.BlockSpec((R, D_H), row),
            pl.BlockSpec((R, 24), row),
            pl.BlockSpec((R, 24), lambda i: (G + i, 0)),
            pl.BlockSpec((R, 24), row),
            pl.BlockSpec((R, 24), lambda i: (G + i, 0)),
            pl.BlockSpec((R, 24), row),
            pl.BlockSpec((R, 24), lambda i: (G + i, 0)),
            pl.BlockSpec((R, 8), row),
            pl.BlockSpec((R, XW), row),
            pl.BlockSpec((R, D_H), row),
            pl.BlockSpec((1, 1, R), lambda i: (i, 0, 0)),
            pl.BlockSpec((D_H, D_H), fix),
            pl.BlockSpec((D_H, D_H), fix),
            pl.BlockSpec((1, D_H), fix),
        ],
        out_specs=[
            pl.BlockSpec((R, D_H), row),
            pl.BlockSpec((R, D_H), row),
            pl.BlockSpec((R, XW), row),
            pl.BlockSpec((N_GRAPHS, XW), fix),
        ],
        out_shape=[
            jax.ShapeDtypeStruct((N_NODES, D_H), f32),
            jax.ShapeDtypeStruct((N_NODES, D_H), f32),
            jax.ShapeDtypeStruct((N_NODES, XW), f32),
            jax.ShapeDtypeStruct((N_GRAPHS, XW), f32),
        ],
    )(H1, T0, T0, T1, T1, T2, T2, inv8, X1, Z, batch3, Wo, Wz, bz)


def _center(X2, batch3, gsums):
    R = _RNODE
    G = N_NODES // R
    f32 = jnp.float32

    def body(x2, b3, gs, o):
        mean = gs[...] * (1.0 / (gs[:, 3:4] + 1e-9))
        b = b3[0, 0, :]
        gids = lax.broadcasted_iota(jnp.int32, (R, N_GRAPHS), 1)
        onehot = (jnp.broadcast_to(b[:, None], (R, N_GRAPHS)) == gids).astype(f32)
        o[...] = x2[...] - jnp.dot(onehot, mean, preferred_element_type=f32)

    return pl.pallas_call(
        body,
        grid=(G,),
        in_specs=[
            pl.BlockSpec((R, XW), lambda i: (i, 0)),
            pl.BlockSpec((1, 1, R), lambda i: (i, 0, 0)),
            pl.BlockSpec((N_GRAPHS, XW), lambda i: (0, 0)),
        ],
        out_specs=pl.BlockSpec((R, XW), lambda i: (i, 0)),
        out_shape=jax.ShapeDtypeStruct((N_NODES, XW), f32),
    )(X2, batch3, gsums)


def kernel(batch, X, H, E_idx, E, Z, params):
    p = params
    f32 = jnp.float32
    src = E_idx[0].astype(jnp.int32)
    dst = E_idx[1].astype(jnp.int32)
    Xp = jnp.pad(X.astype(f32), ((0, 0), (0, XW - 3)))
    batch3 = batch.astype(jnp.int32).reshape(N_NODES // _RNODE, 1, _RNODE)

    z24 = jnp.zeros((1000, 24), f32)
    z16 = jnp.zeros((1000, XW), f32)

    bm1 = p['b_m1'].reshape(1, D_H)
    bm2 = p['b_m2'].reshape(1, D_H)
    Wx8 = jnp.pad(p['W_x'], ((0, 0), (0, 7)))
    bx8 = jnp.pad(p['b_x'].reshape(1, 1), ((0, 0), (0, 7)))
    be = p['b_e'].reshape(1, D_E)
    bh1 = p['b_h1'].reshape(1, D_H)
    bh2 = p['b_h2'].reshape(1, D_H)
    be2 = p['b_e2'].reshape(1, D_E)
    bz = p['b_z'].reshape(1, D_H)
    Web8 = jnp.pad(p['W_eb'], ((0, 0), (0, 8 - HEADS)))
    shead = np.zeros((D_H, 8), np.float32)
    for h in range(HEADS):
        shead[h * D_HEAD:(h + 1) * D_HEAD, h] = 1.0
    Shead = jnp.asarray(shead)
    ehead = np.zeros((HEADS, D_H), np.float32)
    for h in range(HEADS):
        ehead[h, h * D_HEAD:(h + 1) * D_HEAD] = 1.0
    Ehead = jnp.asarray(ehead)

    Hd, Hs, Xs, Xd = _gather_phase_a(H, Xp, src, dst)
    P0, P1, P2, E1 = _edge_mlp(Hd, Hs, Xs, Xd, E, p['W_m1'], bm1, p['W_m2'], bm2,
                               Wx8, bx8, p['W_e'], be)
    S0, S1, S2 = _scatter_add3([P0, P1, P2], dst, 24, z24)
    H1, X1, q, KV, inv8 = _node_update(H, Xp, S0, S1, S2, p['W_h1'], bh1,
                                       p['W_h2'], bh2, p['W_q'], p['W_k'], p['W_v'])
    qd, KVs = _gather_phase_b(q, KV, src, dst)
    logits, gmax = _logits(qd, KVs, E1, Shead, Web8)
    alpha16 = _alpha(logits, gmax)
    D16, = _scatter_add3([alpha16], dst, XW, z16)
    denr = _denr(D16)
    denrd = _gather_rows16(denr, dst)
    Q0, Q1, Q2, E2 = _attn_msgs(alpha16, denrd, KVs, Xs, Xd, E1, Ehead,
                                p['W_e2'], be2)
    T0, T1, T2 = _scatter_add3([Q0, Q1, Q2], dst, 24, z24)
    H2, Z1, X2, gsums = _final_nodes(H1, T0, T1, T2, inv8, X1, Z, batch3,
                                     p['W_o'], p['W_z'], bz)
    Xout = _center(X2, batch3, gsums)
    return (Xout[:, :3], H2, E2, Z1)
